# baseline (speedup 1.0000x reference)
import functools

import jax
import jax.numpy as jnp
from jax import lax
from jax.experimental import pallas as pl
from jax.experimental.pallas import tpu as pltpu
from jax.experimental.pallas import tpu_sc as plsc

N = 10000
E = 320000
D = 128
HD = 2 * D
L = 4
NLIN = 2
EPS = 1e-7

NSUB = 16
CHUNK = 128
NCHUNK = 158
IDXCH = NCHUNK + 2
ESUB = IDXCH * CHUNK
EPADT = NSUB * ESUB
NPAD = 10112
NODES_PER_SUB = NPAD // NSUB


def _sc_edge_body(u, srcp, dstp, coef, tsplat, accout,
                  acc, sidx_a, didx_a, sidx_b, didx_b, rows_a, rows_b,
                  out_a, out_b, coef_v, tv,
                  sem_a, sem_b, ssem_a, ssem_b,
                  isem_sa, isem_da, isem_sb, isem_db):
    c = lax.axis_index("c")
    s = lax.axis_index("s")
    c64 = c * 64
    ebase = s * ESUB
    sbase = c * EPADT + ebase

    pltpu.sync_copy(coef.at[pl.ds(c64, 64)], coef_v)
    pltpu.sync_copy(tsplat, tv)

    zeros16 = jnp.zeros((16,), jnp.float32)

    def zfill(i, _):
        for f in range(8):
            out_a[i, pl.ds(16 * f, 16)] = zeros16
            out_b[i, pl.ds(16 * f, 16)] = zeros16
        return 0

    lax.fori_loop(0, CHUNK, zfill, 0)
    arow = s * NODES_PER_SUB
    for k in range(4):
        pltpu.sync_copy(out_a, acc.at[pl.ds(arow + k * CHUNK, CHUNK)])
    pltpu.sync_copy(out_a.at[pl.ds(0, 120)], acc.at[pl.ds(arow + 512, 120)])
    plsc.subcore_barrier()

    tval = tv[...]
    cvals = [coef_v[pl.ds(16 * f, 16)] for f in range(4)]

    def compute(rows_v, out_v):
        @plsc.parallel_loop(0, CHUNK, 2, unroll=4)
        def _body(e):
            for ee in range(2):
                for f in range(4):
                    v = rows_v[e + ee, pl.ds(16 * f, 16)]
                    r = jnp.maximum(v, 0.0)
                    w = jnp.exp(r * tval + cvals[f])
                    wm = w * (r + EPS)
                    out_v[e + ee, pl.ds(16 * f, 16)] = w
                    out_v[e + ee, pl.ds(64 + 16 * f, 16)] = wm

    def load_idx(arr, j, buf, sem, base):
        pltpu.async_copy(arr.at[pl.ds(base + j * CHUNK, CHUNK)], buf, sem)

    def wait_idx(arr, buf, sem):
        pltpu.make_async_copy(arr.at[pl.ds(ebase, CHUNK)], buf, sem).wait()

    def wait_gather(sidx, rows, sem):
        pltpu.make_async_copy(u.at[sidx], rows, sem).wait()

    def wait_scatter(out_v, didx, sem):
        pltpu.make_async_copy(out_v, acc.at[didx], sem).wait()

    load_idx(srcp, 0, sidx_a, isem_sa, sbase)
    load_idx(srcp, 1, sidx_b, isem_sb, sbase)
    pltpu.sync_copy(dstp.at[pl.ds(ebase + NCHUNK * CHUNK, CHUNK)], didx_a)
    pltpu.sync_copy(dstp.at[pl.ds(ebase + (NCHUNK + 1) * CHUNK, CHUNK)], didx_b)
    pltpu.async_copy(out_a, acc.at[didx_a], ssem_a, add=True)
    pltpu.async_copy(out_b, acc.at[didx_b], ssem_b, add=True)
    wait_idx(srcp, sidx_a, isem_sa)
    pltpu.async_copy(u.at[sidx_a], rows_a, sem_a)
    wait_idx(srcp, sidx_b, isem_sb)
    pltpu.async_copy(u.at[sidx_b], rows_b, sem_b)

    def pipe_body(jj, _):
        j0 = 2 * jj
        j1 = j0 + 1
        j2 = j0 + 2
        j3 = j0 + 3
        wait_gather(sidx_a, rows_a, sem_a)
        load_idx(srcp, j2, sidx_a, isem_sa, sbase)
        wait_scatter(out_a, didx_a, ssem_a)
        load_idx(dstp, j0, didx_a, isem_da, ebase)
        compute(rows_a, out_a)
        wait_idx(dstp, didx_a, isem_da)
        pltpu.async_copy(out_a, acc.at[didx_a], ssem_a, add=True)
        wait_idx(srcp, sidx_a, isem_sa)
        pltpu.async_copy(u.at[sidx_a], rows_a, sem_a)
        wait_gather(sidx_b, rows_b, sem_b)
        load_idx(srcp, j3, sidx_b, isem_sb, sbase)
        wait_scatter(out_b, didx_b, ssem_b)
        load_idx(dstp, j1, didx_b, isem_db, ebase)
        compute(rows_b, out_b)
        wait_idx(dstp, didx_b, isem_db)
        pltpu.async_copy(out_b, acc.at[didx_b], ssem_b, add=True)
        wait_idx(srcp, sidx_b, isem_sb)
        pltpu.async_copy(u.at[sidx_b], rows_b, sem_b)
        return 0

    lax.fori_loop(0, NCHUNK // 2, pipe_body, 0)

    wait_gather(sidx_a, rows_a, sem_a)
    wait_gather(sidx_b, rows_b, sem_b)
    wait_scatter(out_a, didx_a, ssem_a)
    wait_scatter(out_b, didx_b, ssem_b)
    plsc.subcore_barrier()

    for k in range(4):
        r0 = arow + k * 128
        pltpu.sync_copy(acc.at[pl.ds(r0, 128)],
                        accout.at[pl.ds(c * NPAD + r0, 128)])
    pltpu.sync_copy(acc.at[pl.ds(arow + 512, 120)],
                    accout.at[pl.ds(c * NPAD + arow + 512, 120)])


_sc_edge = functools.partial(
    pl.kernel,
    out_type=jax.ShapeDtypeStruct((2 * NPAD, D), jnp.float32),
    mesh=plsc.VectorSubcoreMesh(core_axis_name="c", subcore_axis_name="s"),
    compiler_params=pltpu.CompilerParams(use_tc_tiling_on_sc=False),
    scratch_types=[
        pltpu.VMEM_SHARED((NPAD, D), jnp.float32),
        pltpu.VMEM((CHUNK,), jnp.int32),
        pltpu.VMEM((CHUNK,), jnp.int32),
        pltpu.VMEM((CHUNK,), jnp.int32),
        pltpu.VMEM((CHUNK,), jnp.int32),
        pltpu.VMEM((CHUNK, 64), jnp.float32),
        pltpu.VMEM((CHUNK, 64), jnp.float32),
        pltpu.VMEM((CHUNK, D), jnp.float32),
        pltpu.VMEM((CHUNK, D), jnp.float32),
        pltpu.VMEM((64,), jnp.float32),
        pltpu.VMEM((16,), jnp.float32),
        pltpu.SemaphoreType.DMA,
        pltpu.SemaphoreType.DMA,
        pltpu.SemaphoreType.DMA,
        pltpu.SemaphoreType.DMA,
        pltpu.SemaphoreType.DMA,
        pltpu.SemaphoreType.DMA,
        pltpu.SemaphoreType.DMA,
        pltpu.SemaphoreType.DMA,
    ],
)(_sc_edge_body)


def _prep0_body(x_ref, hs_ref, mx_ref):
    x = x_ref[...]
    hs_ref[pl.ds(0, N), :] = x[:, 0:64]
    hs_ref[pl.ds(N, N), :] = x[:, 64:128]
    mx = jnp.max(jax.nn.relu(x), axis=0, keepdims=True) + EPS
    mx_ref[...] = jnp.broadcast_to(mx, (8, D))


def _gnprep_body(h_ref, g_ref, b_ref, a_ref, u_ref, hs_ref, mx_ref):
    h = h_ref[...]
    mu = jnp.mean(h, axis=0)
    hc = h - a_ref[...][0] * mu
    var = jnp.mean(hc * hc, axis=0)
    u = jax.nn.relu(g_ref[...][0] * hc / jnp.sqrt(var + 1e-5) + b_ref[...][0])
    u_ref[...] = u
    hs_ref[pl.ds(0, N), :] = u[:, 0:64]
    hs_ref[pl.ds(N, N), :] = u[:, 64:128]
    mx_ref[...] = jnp.broadcast_to(jnp.max(u, axis=0, keepdims=True) + EPS, (8, D))


def _mlp_body(u_ref, acc_ref, hres_ref, w1_ref, b1_ref, g_ref, b_ref,
              w2_ref, b2_ref, o_ref):
    sfull = jnp.concatenate(
        [acc_ref[pl.ds(0, N), 0:64], acc_ref[pl.ds(NPAD, N), 0:64]], axis=1)
    num = jnp.concatenate(
        [acc_ref[pl.ds(0, N), 64:128], acc_ref[pl.ds(NPAD, N), 64:128]], axis=1)
    aggr = num / (sfull + 1e-16)
    z = u_ref[...] + aggr
    z = jnp.dot(z, w1_ref[...], preferred_element_type=jnp.float32) + b1_ref[...][0]
    mu = jnp.mean(z, axis=0)
    var = jnp.mean(z * z, axis=0) - mu * mu
    z = (z - mu) / jnp.sqrt(var + 1e-5) * g_ref[...][0] + b_ref[...][0]
    z = jax.nn.relu(z)
    o_ref[...] = (
        jnp.dot(z, w2_ref[...], preferred_element_type=jnp.float32)
        + b2_ref[...][0] + hres_ref[...]
    )


def _head_body(h_ref, w0_ref, b0_ref, w1_ref, b1_ref, o_ref):
    z = jax.nn.relu(
        jnp.dot(h_ref[...], w0_ref[...], preferred_element_type=jnp.float32)
        + b0_ref[...][0])
    o_ref[...] = (
        jnp.dot(z, w1_ref[...], preferred_element_type=jnp.float32) + b1_ref[...][0])


def _row(v):
    return v.reshape(1, -1)


def kernel(x, edge_index, t, W1, b1, bn_g, bn_b, W2, b2, gn_g, gn_b, gn_a, LW, Lb):
    order = jnp.argsort(edge_index[0])
    edge_index = edge_index[:, order]
    npad = ESUB - E // NSUB
    srcp1 = jnp.pad(edge_index[0].reshape(NSUB, E // NSUB),
                    ((0, 0), (0, npad))).reshape(EPADT)
    srcp = jnp.concatenate([srcp1, srcp1 + N])
    dstp = jnp.pad(edge_index[1].reshape(NSUB, E // NSUB),
                   ((0, 0), (0, npad)), constant_values=N).reshape(EPADT)

    prep0 = pl.pallas_call(
        _prep0_body,
        out_shape=[
            jax.ShapeDtypeStruct((2 * N, 64), jnp.float32),
            jax.ShapeDtypeStruct((8, D), jnp.float32),
        ],
    )
    gnprep = pl.pallas_call(
        _gnprep_body,
        out_shape=[
            jax.ShapeDtypeStruct((N, D), jnp.float32),
            jax.ShapeDtypeStruct((2 * N, 64), jnp.float32),
            jax.ShapeDtypeStruct((8, D), jnp.float32),
        ],
    )
    mlp = pl.pallas_call(
        _mlp_body,
        out_shape=jax.ShapeDtypeStruct((N, D), jnp.float32),
    )
    head = pl.pallas_call(
        _head_body,
        out_shape=jax.ShapeDtypeStruct((N, D), jnp.float32),
    )

    hsrc, mx8 = prep0(x)
    u = x
    hres = jnp.zeros((N, D), jnp.float32)
    h = None
    for i in range(L):
        t_i = t[i]
        mx = mx8[0]
        shift = jnp.maximum(t_i * mx, t_i * EPS)
        coef = t_i * EPS - shift
        tsplat = jnp.full((16,), t_i, jnp.float32)
        acc = _sc_edge(hsrc, srcp, dstp, coef, tsplat)
        h = mlp(u, acc, hres, W1[i], _row(b1[i]), _row(bn_g[i]), _row(bn_b[i]),
                W2[i], _row(b2[i]))
        if i < L - 1:
            u, hsrc, mx8 = gnprep(h, _row(gn_g[i]), _row(gn_b[i]), _row(gn_a[i]))
            hres = h
    return head(h, LW[0], _row(Lb[0]), LW[1], _row(Lb[1]))


# speedup vs baseline: 2.0126x; 2.0126x over previous
import functools

import jax
import jax.numpy as jnp
from jax import lax
from jax.experimental import pallas as pl
from jax.experimental.pallas import tpu as pltpu
from jax.experimental.pallas import tpu_sc as plsc

N = 10000
E = 320000
D = 128
HD = 2 * D
L = 4
NLIN = 2
EPS = 1e-7

NSUB = 16
CHUNK = 128
NCHUNK = 158
IDXCH = NCHUNK + 2
ESUB = IDXCH * CHUNK
EPADT = NSUB * ESUB
NPAD = 10112
NODES_PER_SUB = NPAD // NSUB


def _sc_edge_body(u, srcp, dstp, coef, tsplat, accout,
                  acc, sidx_a, didx_a, sidx_b, didx_b, rows_a, rows_b,
                  out_a, out_b, coef_v, tv,
                  sem_a, sem_b, ssem_a, ssem_b,
                  isem_sa, isem_da, isem_sb, isem_db):
    c = lax.axis_index("c")
    s = lax.axis_index("s")
    c64 = c * 64
    ebase = s * ESUB
    sbase = c * EPADT + ebase

    pltpu.sync_copy(coef.at[pl.ds(c64, 64)], coef_v)
    pltpu.sync_copy(tsplat, tv)

    zeros16 = jnp.zeros((16,), jnp.float32)

    def zfill(i, _):
        for f in range(8):
            out_a[i, pl.ds(16 * f, 16)] = zeros16
            out_b[i, pl.ds(16 * f, 16)] = zeros16
        return 0

    lax.fori_loop(0, CHUNK, zfill, 0)
    arow = s * NODES_PER_SUB
    for k in range(4):
        pltpu.sync_copy(out_a, acc.at[pl.ds(arow + k * CHUNK, CHUNK)])
    pltpu.sync_copy(out_a.at[pl.ds(0, 120)], acc.at[pl.ds(arow + 512, 120)])
    plsc.subcore_barrier()

    tval = tv[...]
    cvals = [coef_v[pl.ds(16 * f, 16)] for f in range(4)]

    def compute(rows_v, out_v):
        @plsc.parallel_loop(0, CHUNK, 2, unroll=4)
        def _body(e):
            for ee in range(2):
                for f in range(2):
                    w32 = rows_v[e + ee, pl.ds(16 * f, 16)]
                    va = plsc.bitcast(lax.shift_left(w32, 16), jnp.float32)
                    vbb = plsc.bitcast(
                        lax.bitwise_and(w32, jnp.int32(-65536)), jnp.float32)
                    for g, vv in ((2 * f, va), (2 * f + 1, vbb)):
                        r = jnp.maximum(vv, 0.0)
                        w = jnp.exp(r * tval + cvals[g])
                        wm = w * (r + EPS)
                        out_v[e + ee, pl.ds(16 * g, 16)] = w
                        out_v[e + ee, pl.ds(64 + 16 * g, 16)] = wm

    def load_idx(arr, j, buf, sem, base):
        pltpu.async_copy(arr.at[pl.ds(base + j * CHUNK, CHUNK)], buf, sem)

    def wait_idx(arr, buf, sem):
        pltpu.make_async_copy(arr.at[pl.ds(ebase, CHUNK)], buf, sem).wait()

    def wait_gather(sidx, rows, sem):
        pltpu.make_async_copy(u.at[sidx], rows, sem).wait()

    def wait_scatter(out_v, didx, sem):
        pltpu.make_async_copy(out_v, acc.at[didx], sem).wait()

    load_idx(srcp, 0, sidx_a, isem_sa, sbase)
    load_idx(srcp, 1, sidx_b, isem_sb, sbase)
    pltpu.sync_copy(dstp.at[pl.ds(ebase + NCHUNK * CHUNK, CHUNK)], didx_a)
    pltpu.sync_copy(dstp.at[pl.ds(ebase + (NCHUNK + 1) * CHUNK, CHUNK)], didx_b)
    pltpu.async_copy(out_a, acc.at[didx_a], ssem_a, add=True)
    pltpu.async_copy(out_b, acc.at[didx_b], ssem_b, add=True)
    wait_idx(srcp, sidx_a, isem_sa)
    pltpu.async_copy(u.at[sidx_a], rows_a, sem_a)
    wait_idx(srcp, sidx_b, isem_sb)
    pltpu.async_copy(u.at[sidx_b], rows_b, sem_b)

    def pipe_body(jj, _):
        j0 = 2 * jj
        j1 = j0 + 1
        j2 = j0 + 2
        j3 = j0 + 3
        wait_gather(sidx_a, rows_a, sem_a)
        load_idx(srcp, j2, sidx_a, isem_sa, sbase)
        wait_scatter(out_a, didx_a, ssem_a)
        load_idx(dstp, j0, didx_a, isem_da, ebase)
        compute(rows_a, out_a)
        wait_idx(dstp, didx_a, isem_da)
        pltpu.async_copy(out_a, acc.at[didx_a], ssem_a, add=True)
        wait_idx(srcp, sidx_a, isem_sa)
        pltpu.async_copy(u.at[sidx_a], rows_a, sem_a)
        wait_gather(sidx_b, rows_b, sem_b)
        load_idx(srcp, j3, sidx_b, isem_sb, sbase)
        wait_scatter(out_b, didx_b, ssem_b)
        load_idx(dstp, j1, didx_b, isem_db, ebase)
        compute(rows_b, out_b)
        wait_idx(dstp, didx_b, isem_db)
        pltpu.async_copy(out_b, acc.at[didx_b], ssem_b, add=True)
        wait_idx(srcp, sidx_b, isem_sb)
        pltpu.async_copy(u.at[sidx_b], rows_b, sem_b)
        return 0

    lax.fori_loop(0, NCHUNK // 2, pipe_body, 0)

    wait_gather(sidx_a, rows_a, sem_a)
    wait_gather(sidx_b, rows_b, sem_b)
    wait_scatter(out_a, didx_a, ssem_a)
    wait_scatter(out_b, didx_b, ssem_b)
    plsc.subcore_barrier()

    for k in range(4):
        r0 = arow + k * 128
        pltpu.sync_copy(acc.at[pl.ds(r0, 128)],
                        accout.at[pl.ds(c * NPAD + r0, 128)])
    pltpu.sync_copy(acc.at[pl.ds(arow + 512, 120)],
                    accout.at[pl.ds(c * NPAD + arow + 512, 120)])


_sc_edge = functools.partial(
    pl.kernel,
    out_type=jax.ShapeDtypeStruct((2 * NPAD, D), jnp.float32),
    mesh=plsc.VectorSubcoreMesh(core_axis_name="c", subcore_axis_name="s"),
    compiler_params=pltpu.CompilerParams(use_tc_tiling_on_sc=False,
                                        needs_layout_passes=False),
    scratch_types=[
        pltpu.VMEM_SHARED((NPAD, D), jnp.float32),
        pltpu.VMEM((CHUNK,), jnp.int32),
        pltpu.VMEM((CHUNK,), jnp.int32),
        pltpu.VMEM((CHUNK,), jnp.int32),
        pltpu.VMEM((CHUNK,), jnp.int32),
        pltpu.VMEM((CHUNK, 32), jnp.int32),
        pltpu.VMEM((CHUNK, 32), jnp.int32),
        pltpu.VMEM((CHUNK, D), jnp.float32),
        pltpu.VMEM((CHUNK, D), jnp.float32),
        pltpu.VMEM((64,), jnp.float32),
        pltpu.VMEM((16,), jnp.float32),
        pltpu.SemaphoreType.DMA,
        pltpu.SemaphoreType.DMA,
        pltpu.SemaphoreType.DMA,
        pltpu.SemaphoreType.DMA,
        pltpu.SemaphoreType.DMA,
        pltpu.SemaphoreType.DMA,
        pltpu.SemaphoreType.DMA,
        pltpu.SemaphoreType.DMA,
    ],
)(_sc_edge_body)


def _prep0_body(x_ref, mx_ref):
    mx = jnp.max(jax.nn.relu(x_ref[...]), axis=0, keepdims=True) + EPS
    mx_ref[...] = jnp.broadcast_to(mx, (8, D))


def _gnprep_body(h_ref, g_ref, b_ref, a_ref, u_ref, mx_ref):
    h = h_ref[...]
    mu = jnp.mean(h, axis=0)
    hc = h - a_ref[...][0] * mu
    var = jnp.mean(hc * hc, axis=0)
    u = jax.nn.relu(g_ref[...][0] * hc / jnp.sqrt(var + 1e-5) + b_ref[...][0])
    u_ref[...] = u
    mx_ref[...] = jnp.broadcast_to(jnp.max(u, axis=0, keepdims=True) + EPS, (8, D))


def _mlp_body(u_ref, acc_ref, hres_ref, w1_ref, b1_ref, g_ref, b_ref,
              w2_ref, b2_ref, o_ref):
    sfull = jnp.concatenate(
        [acc_ref[pl.ds(0, N), 0:64], acc_ref[pl.ds(NPAD, N), 0:64]], axis=1)
    num = jnp.concatenate(
        [acc_ref[pl.ds(0, N), 64:128], acc_ref[pl.ds(NPAD, N), 64:128]], axis=1)
    aggr = num / (sfull + 1e-16)
    z = u_ref[...] + aggr
    z = jnp.dot(z, w1_ref[...], preferred_element_type=jnp.float32) + b1_ref[...][0]
    mu = jnp.mean(z, axis=0)
    var = jnp.mean(z * z, axis=0) - mu * mu
    z = (z - mu) / jnp.sqrt(var + 1e-5) * g_ref[...][0] + b_ref[...][0]
    z = jax.nn.relu(z)
    o_ref[...] = (
        jnp.dot(z, w2_ref[...], preferred_element_type=jnp.float32)
        + b2_ref[...][0] + hres_ref[...]
    )


def _head_body(h_ref, w0_ref, b0_ref, w1_ref, b1_ref, o_ref):
    z = jax.nn.relu(
        jnp.dot(h_ref[...], w0_ref[...], preferred_element_type=jnp.float32)
        + b0_ref[...][0])
    o_ref[...] = (
        jnp.dot(z, w1_ref[...], preferred_element_type=jnp.float32) + b1_ref[...][0])


_LO = [b + i for b in range(0, D, 32) for i in range(16)]
_HI = [b + 16 + i for b in range(0, D, 32) for i in range(16)]


def _pack_table(u):
    ub = u.astype(jnp.bfloat16)
    lo = lax.bitcast_convert_type(ub[:, jnp.array(_LO)], jnp.uint16)
    hi = lax.bitcast_convert_type(ub[:, jnp.array(_HI)], jnp.uint16)
    word = (hi.astype(jnp.uint32) << 16) | lo.astype(jnp.uint32)
    word = lax.bitcast_convert_type(word, jnp.int32)
    return jnp.concatenate([word[:, :32], word[:, 32:]], axis=0)


def _row(v):
    return v.reshape(1, -1)


def kernel(x, edge_index, t, W1, b1, bn_g, bn_b, W2, b2, gn_g, gn_b, gn_a, LW, Lb):
    npad = ESUB - E // NSUB
    srcp1 = jnp.pad(edge_index[0].reshape(NSUB, E // NSUB),
                    ((0, 0), (0, npad))).reshape(EPADT)
    srcp = jnp.concatenate([srcp1, srcp1 + N])
    dstp = jnp.pad(edge_index[1].reshape(NSUB, E // NSUB),
                   ((0, 0), (0, npad)), constant_values=N).reshape(EPADT)

    prep0 = pl.pallas_call(
        _prep0_body,
        out_shape=jax.ShapeDtypeStruct((8, D), jnp.float32),
    )
    gnprep = pl.pallas_call(
        _gnprep_body,
        out_shape=[
            jax.ShapeDtypeStruct((N, D), jnp.float32),
            jax.ShapeDtypeStruct((8, D), jnp.float32),
        ],
    )
    mlp = pl.pallas_call(
        _mlp_body,
        out_shape=jax.ShapeDtypeStruct((N, D), jnp.float32),
    )
    head = pl.pallas_call(
        _head_body,
        out_shape=jax.ShapeDtypeStruct((N, D), jnp.float32),
    )

    mx8 = prep0(x)
    u = x
    hres = jnp.zeros((N, D), jnp.float32)
    h = None
    for i in range(L):
        t_i = t[i]
        mx = mx8[0]
        shift = jnp.maximum(t_i * mx, t_i * EPS)
        coef = t_i * EPS - shift
        tsplat = jnp.full((16,), t_i, jnp.float32)
        htab = _pack_table(u)
        acc = _sc_edge(htab, srcp, dstp, coef, tsplat)
        h = mlp(u, acc, hres, W1[i], _row(b1[i]), _row(bn_g[i]), _row(bn_b[i]),
                W2[i], _row(b2[i]))
        if i < L - 1:
            u, mx8 = gnprep(h, _row(gn_g[i]), _row(gn_b[i]), _row(gn_a[i]))
            hres = h
    return head(h, LW[0], _row(Lb[0]), LW[1], _row(Lb[1]))
